# baseline (speedup 1.0000x reference)
import functools

import jax
import jax.numpy as jnp
import numpy as np
from jax import lax
from jax.experimental import pallas as pl
from jax.experimental.pallas import tpu as pltpu
from jax.experimental.pallas import tpu_sc as plsc

P = 65536
SD = 64
CD = 32
OD = 128
H = 256
ALPHA = 0.5

_TINY = np.float32(np.finfo(np.float32).tiny)
_KS1 = np.int32(7)
_KS2 = np.int32(0x1BD11BDA ^ 7)
_ONEBITS = np.int32(0x3F800000)


_BLK = 1024
_NBLK = P // _BLK


def _mlp_body(x_ref, lw_ref, noise_ref, obs_ref, ctrl_ref,
              w1a_ref, w1b_ref, b1_ref, w2_ref, b2_ref,
              v1a_ref, v1b_ref, c1_ref, v2_ref, c2_ref,
              spred_ref, lwp_ref, r_ref, logw_ref):
    x = x_ref[...]
    b1p = b1_ref[...] + jnp.dot(ctrl_ref[...], w1b_ref[...],
                                preferred_element_type=jnp.float32)
    h = jnp.tanh(jnp.dot(x, w1a_ref[...],
                         preferred_element_type=jnp.float32) + b1p)
    delta = jnp.dot(h, w2_ref[...],
                    preferred_element_type=jnp.float32) + b2_ref[...]
    spred = x + delta + noise_ref[...]
    spred_ref[...] = spred
    c1p = c1_ref[...] + jnp.dot(obs_ref[...], v1a_ref[...],
                                preferred_element_type=jnp.float32)
    m = jnp.tanh(jnp.dot(spred, v1b_ref[...],
                         preferred_element_type=jnp.float32) + c1p)
    loglik = jnp.dot(m, v2_ref[...],
                     preferred_element_type=jnp.float32) + c2_ref[...]
    lwp = lw_ref[...] + loglik
    lwp_ref[...] = lwp
    w = ALPHA * jnp.exp(lwp) + np.float32((1.0 - ALPHA) / P)
    r_ref[...] = 1.0 / w
    logw_ref[...] = jnp.log(w)


def _run_mlp(states_prev, lw_prev, noise, observation, control,
             W1, b1, W2, b2, V1, c1, V2, c2):
    f32 = jnp.float32
    grid = (_NBLK,)
    blk = lambda i: (i, 0)
    const = lambda i: (0, 0)
    return pl.pallas_call(
        _mlp_body,
        grid=grid,
        in_specs=[
            pl.BlockSpec((_BLK, SD), blk),
            pl.BlockSpec((_BLK, 1), blk),
            pl.BlockSpec((_BLK, SD), blk),
            pl.BlockSpec((1, OD), const),
            pl.BlockSpec((1, CD), const),
            pl.BlockSpec((SD, H), const),
            pl.BlockSpec((CD, H), const),
            pl.BlockSpec((1, H), const),
            pl.BlockSpec((H, SD), const),
            pl.BlockSpec((1, SD), const),
            pl.BlockSpec((OD, H), const),
            pl.BlockSpec((SD, H), const),
            pl.BlockSpec((1, H), const),
            pl.BlockSpec((H, 1), const),
            pl.BlockSpec((1, 1), const),
        ],
        out_specs=[
            pl.BlockSpec((_BLK, SD), blk),
            pl.BlockSpec((_BLK, 1), blk),
            pl.BlockSpec((_BLK, 1), blk),
            pl.BlockSpec((_BLK, 1), blk),
        ],
        out_shape=[
            jax.ShapeDtypeStruct((P, SD), f32),
            jax.ShapeDtypeStruct((P, 1), f32),
            jax.ShapeDtypeStruct((P, 1), f32),
            jax.ShapeDtypeStruct((P, 1), f32),
        ],
    )(states_prev, lw_prev, noise,
      observation.reshape(1, OD), control.reshape(1, CD),
      W1[:SD], W1[SD:], b1.reshape(1, H), W2, b2.reshape(1, SD),
      V1[:OD], V1[OD:], c1.reshape(1, H), V2, c2.reshape(1, 1))


def _finalize_body(lwp_ref, logw_ref, lw_out_ref, bi_ref):
    lwp = lwp_ref[...]
    rows, cols = lwp.shape
    pidx = (lax.broadcasted_iota(jnp.int32, lwp.shape, 0) * cols
            + lax.broadcasted_iota(jnp.int32, lwp.shape, 1))
    mx = jnp.max(lwp)
    cand = jnp.where(lwp == mx, pidx, np.int32(P))
    bi_ref[0, 0] = jnp.min(cand)
    lw1 = lwp - logw_ref[...]
    m = jnp.max(lw1)
    lse = jnp.log(jnp.sum(jnp.exp(lw1 - m))) + m
    lw_out_ref[...] = lw1 - lse


def _run_finalize(lwp, logw):
    return pl.pallas_call(
        _finalize_body,
        out_specs=[
            pl.BlockSpec(memory_space=pltpu.VMEM),
            pl.BlockSpec(memory_space=pltpu.SMEM),
        ],
        out_shape=[
            jax.ShapeDtypeStruct((P // 128, 128), jnp.float32),
            jax.ShapeDtypeStruct((1, 1), jnp.int32),
        ],
        in_specs=[
            pl.BlockSpec(memory_space=pltpu.VMEM),
            pl.BlockSpec(memory_space=pltpu.VMEM),
        ],
    )(lwp.reshape(P // 128, 128), logw.reshape(P // 128, 128))


_ROWS = 8
_NPROG = P // _ROWS
_JBLK = 2048
_STEPS = P // _JBLK


def _rotl(x, r):
    return lax.shift_left(x, np.int32(r)) | lax.shift_right_logical(
        x, np.int32(32 - r))


def _tf_round(x0, x1, r):
    x0 = x0 + x1
    x1 = x0 ^ _rotl(x1, r)
    return x0, x1


def _threefry_bits(f):
    x1 = f + _KS1
    x0 = x1
    x1 = x0 ^ _rotl(x1, 13)
    x0, x1 = _tf_round(x0, x1, 15)
    x0, x1 = _tf_round(x0, x1, 26)
    x0, x1 = _tf_round(x0, x1, 6)
    x0 = x0 + _KS1
    x1 = x1 + np.int32(_KS2 + 1)
    for r in (17, 29, 16, 24):
        x0, x1 = _tf_round(x0, x1, r)
    x0 = x0 + _KS2
    x1 = x1 + np.int32(2)
    for r in (13, 15, 26, 6):
        x0, x1 = _tf_round(x0, x1, r)
    x1 = x1 + np.int32(7 + 3)
    for r in (17, 29, 16, 24):
        x0, x1 = _tf_round(x0, x1, r)
    x0 = x0 + _KS1
    x1 = x1 + np.int32(_KS2 + 4)
    for r in (13, 15, 26, 6):
        x0, x1 = _tf_round(x0, x1, r)
    x0 = x0 + _KS2
    x1 = x1 + np.int32(5)
    return x0 ^ x1


def _cat_body(r_ref, out_ref):
    g = pl.program_id(0)
    row_iota = lax.broadcasted_iota(jnp.int32, (_ROWS, _JBLK), 0)
    lane_iota = lax.broadcasted_iota(jnp.int32, (_ROWS, _JBLK), 1)
    base = (g * np.int32(_ROWS * P) + row_iota * np.int32(P)) + lane_iota

    def step(t, carry):
        minval, minstep = carry
        f = base + t * np.int32(_JBLK)
        bits = _threefry_bits(f)
        fb = lax.shift_right_logical(bits, np.int32(9)) | _ONEBITS
        fl = lax.bitcast_convert_type(fb, jnp.float32) - np.float32(1.0)
        e = -jnp.log(jnp.maximum(fl, _TINY))
        score = e * r_ref[pl.ds(t, 1), :]
        upd = score < minval
        minval = jnp.minimum(score, minval)
        minstep = jnp.where(upd, t, minstep)
        return minval, minstep

    minval, minstep = lax.fori_loop(
        0, _STEPS, step,
        (jnp.full((_ROWS, _JBLK), np.float32(np.inf), jnp.float32),
         jnp.zeros((_ROWS, _JBLK), jnp.int32)))
    minidx = minstep * np.int32(_JBLK) + lane_iota
    rowmin = jnp.min(minval, axis=1, keepdims=True)
    cand = jnp.where(minval == rowmin, minidx, np.int32(P))
    rowidx = jnp.min(cand, axis=1, keepdims=True)
    out_ref[...] = jnp.broadcast_to(rowidx, (1, _ROWS, 128))


def _run_categorical(r):
    out = pl.pallas_call(
        _cat_body,
        grid=(_NPROG,),
        in_specs=[pl.BlockSpec((_STEPS, _JBLK), lambda g: (0, 0))],
        out_specs=pl.BlockSpec((1, _ROWS, 128), lambda g: (g, 0, 0)),
        out_shape=jax.ShapeDtypeStruct((_NPROG, _ROWS, 128), jnp.int32),
    )(r.reshape(_STEPS, _JBLK))
    return out[:, :, 0].reshape(P)


_NW = 32
_CHUNK = 128
_BPW_CHUNKS = 17
_BPW = _CHUNK * _BPW_CHUNKS
_BGATHER = _NW * _BPW

@functools.cache
def _make_sc_gather():
    mesh = plsc.VectorSubcoreMesh(core_axis_name="c", subcore_axis_name="s")

    @functools.partial(
        pl.kernel,
        out_type=jax.ShapeDtypeStruct((_BGATHER, SD), jnp.float32),
        mesh=mesh,
        scratch_types=[
            pltpu.VMEM((_CHUNK,), jnp.int32),
            pltpu.VMEM((_CHUNK, SD), jnp.float32),
            pltpu.SemaphoreType.DMA,
        ],
        compiler_params=pltpu.CompilerParams(use_tc_tiling_on_sc=False),
    )
    def _sc_gather(table_hbm, idx_hbm, out_hbm, idx_v, rows_v, sem):
        wid = lax.axis_index("s") * 2 + lax.axis_index("c")
        base = wid * _BPW
        for c in range(_BPW_CHUNKS):
            off = base + c * _CHUNK
            pltpu.sync_copy(idx_hbm.at[pl.ds(off, _CHUNK)], idx_v)
            pltpu.async_copy(table_hbm.at[idx_v], rows_v, sem).wait()
            pltpu.sync_copy(rows_v, out_hbm.at[pl.ds(off, _CHUNK)])

    return _sc_gather


def kernel(states_prev, log_weights_prev, observation, control,
           W1, b1, W2, b2, V1, c1, V2, c2):
    noise = jax.random.normal(jax.random.key(42), (P, SD),
                              dtype=jnp.float32) * np.float32(0.1)
    spred, lwp, r, logw = _run_mlp(
        states_prev, log_weights_prev.reshape(P, 1), noise,
        observation, control, W1, b1, W2, b2, V1, c1, V2, c2)
    lw_out, best_idx = _run_finalize(lwp, logw)
    indices = _run_categorical(r)
    idx_ext = jnp.concatenate(
        [indices, jnp.broadcast_to(best_idx.reshape(1), (_BGATHER - P,))])
    gathered = _make_sc_gather()(spred, idx_ext)
    best_state = gathered[P]
    states = gathered[:P]
    return best_state, states, lw_out.reshape(P)
